# baseline (speedup 1.0000x reference)
import functools

import jax
import jax.numpy as jnp
from jax import lax
from jax.experimental import pallas as pl
from jax.experimental.pallas import tpu as pltpu
from jax.experimental.pallas import tpu_sc as plsc

_B, _C, _J = 2, 12, 23
_H, _W = 256, 320
_HW = _H * _W
_NWORDS = _HW // 2
_G = 40
_G3 = _G ** 3
_HALF = _G // 2
_SPACING = 2.0
_OFFSET = -100.0
_NW = 32
_NTASK = _B * _J * 2
_TPTS = _G3 // 2
_CHUNK = 8000
_NCHUNK = _TPTS // _CHUNK
_ITERS = _CHUNK // 16
_CSPLIT = 6


def _make_sc_call(cam_lo, ncams, has_init, do_scale):
    mesh = plsc.VectorSubcoreMesh(core_axis_name="c", subcore_axis_name="s")
    nstep = ncams * _NCHUNK

    @functools.partial(
        pl.kernel,
        out_type=jax.ShapeDtypeStruct((_B * _J * _G3,), jnp.float32),
        mesh=mesh,
        compiler_params=pltpu.CompilerParams(
            needs_layout_passes=False, use_tc_tiling_on_sc=False),
        scratch_types=[
            pltpu.VMEM((_H // 2, _W), jnp.int32),
            pltpu.VMEM((_H // 2, _W), jnp.int32),
            pltpu.VMEM((_TPTS,), jnp.float32),
            pltpu.VMEM((_CHUNK,), jnp.int32),
            pltpu.VMEM((_CHUNK,), jnp.int32),
            pltpu.SemaphoreType.DMA,
            pltpu.SemaphoreType.DMA,
            pltpu.SemaphoreType.DMA,
            pltpu.SemaphoreType.DMA,
        ],
    )
    def run(*args):
        if has_init:
            hm_hbm, idx_hbm, init_hbm, out_hbm = args[:4]
            rest = args[4:]
        else:
            hm_hbm, idx_hbm, out_hbm = args[:3]
            rest = args[3:]
        (plane_v0, plane_v1, acc_v, idx_v0, idx_v1,
         psem0, psem1, isem0, isem1) = rest
        wid = lax.axis_index("s") * 2 + lax.axis_index("c")
        planes = (plane_v0, plane_v1)
        idxs = (idx_v0, idx_v1)
        psems = (psem0, psem1)
        isems = (isem0, isem1)

        def task(t):
            b = t // (_J * 2)
            rem = t - b * (_J * 2)
            j = rem // 2
            h = rem - j * 2
            out_base = (b * _J + j) * _G3 + h * _TPTS

            def plane_copy(c):
                return pltpu.make_async_copy(
                    hm_hbm.at[b, c, j], planes[c % 2], psems[c % 2])

            def idx_copy(s):
                c, k = divmod(s, _NCHUNK)
                base = (b * _C + cam_lo + c) * _G3 + h * _TPTS + k * _CHUNK
                return pltpu.make_async_copy(
                    idx_hbm.at[pl.ds(base, _CHUNK)],
                    idxs[s % 2], isems[s % 2])

            plane_copy(0).start()
            idx_copy(0).start()
            if has_init:
                pltpu.sync_copy(init_hbm.at[pl.ds(out_base, _TPTS)], acc_v)
            else:
                @plsc.parallel_loop(0, _TPTS // 16, unroll=4)
                def _zero(i):
                    acc_v[pl.ds(i * 16, 16)] = jnp.zeros((16,), jnp.float32)

            for s in range(nstep):
                c, k = divmod(s, _NCHUNK)
                if s + 1 < nstep:
                    idx_copy(s + 1).start()
                if k == 0:
                    plane_copy(c).wait()
                    if c + 1 < ncams:
                        plane_copy(c + 1).start()
                idx_copy(s).wait()
                pbuf = planes[c % 2]
                ibuf = idxs[s % 2]

                @plsc.parallel_loop(0, _ITERS, unroll=8)
                def _gather(i):
                    iv = ibuf[pl.ds(i * 16, 16)]
                    qi = iv & 511
                    ri = (iv >> 9) & 255
                    in_hi = iv >= (1 << 17)
                    w = plsc.load_gather(pbuf, [ri, qi])
                    hi = w & jnp.int32(-65536)
                    lo = w << 16
                    bits = jnp.where(in_hi, hi, lo)
                    val = plsc.bitcast(bits, jnp.float32)
                    plsc.addupdate(
                        acc_v.at[pl.ds(k * _CHUNK + i * 16, 16)], val)

            if do_scale:
                @plsc.parallel_loop(0, _TPTS // 16, unroll=4)
                def _scale(i):
                    sl = pl.ds(i * 16, 16)
                    acc_v[sl] = acc_v[sl] * jnp.float32(1.0 / _C)

            pltpu.sync_copy(acc_v, out_hbm.at[pl.ds(out_base, _TPTS)])

        def rounds(r, carry):
            t = wid + r * _NW

            @pl.when(t < _NTASK)
            def _():
                task(t)

            return carry

        lax.fori_loop(0, 3, rounds, 0)

    return run


def _pack(hm):
    u = lax.bitcast_convert_type(hm, jnp.uint32)
    b16 = (u + jnp.uint32(0x7FFF) + ((u >> 16) & jnp.uint32(1))) >> 16
    wlo = b16[:, :, :, : _H // 2, :]
    whi = b16[:, :, :, _H // 2 :, :]
    return lax.bitcast_convert_type(wlo | (whi << 16), jnp.int32)


def kernel(heatmaps, center, reproLookup):
    cidx = ((center - _OFFSET) / _SPACING).astype(jnp.int32)
    starts = cidx - _HALF

    def slice_b(s):
        return lax.dynamic_slice(
            reproLookup, (jnp.int32(0), s[0], s[1], s[2]), (_C, _G, _G, _G))

    sub_p = jax.vmap(slice_b)(starts).reshape(_B * _C * _G3)
    in_hi = sub_p >= _NWORDS
    wi = sub_p - jnp.where(in_hi, _NWORDS, 0)
    sub_idx = (wi // _W) * 512 + wi % _W + jnp.where(in_hi, 1 << 17, 0)

    part = None
    for lo in range(0, _C, _CSPLIT):
        hm_words = _pack(heatmaps[:, lo:lo + _CSPLIT])
        last = lo + _CSPLIT >= _C
        if part is None:
            part = _make_sc_call(lo, _CSPLIT, False, last)(hm_words, sub_idx)
        else:
            part = _make_sc_call(lo, _CSPLIT, True, last)(
                hm_words, sub_idx, part)
    return part.reshape(_B, _J, _G, _G, _G)


# speedup vs baseline: 1.4079x; 1.4079x over previous
import functools

import jax
import jax.numpy as jnp
from jax import lax
from jax.experimental import pallas as pl
from jax.experimental.pallas import tpu as pltpu
from jax.experimental.pallas import tpu_sc as plsc

_B, _C, _J = 2, 12, 23
_H, _W = 256, 320
_HW = _H * _W
_NWORDS = _HW // 2
_G = 40
_G3 = _G ** 3
_HALF = _G // 2
_SPACING = 2.0
_OFFSET = -100.0
_NW = 32
_NTASK = _B * _J * 2
_TPTS = _G3 // 2
_CHUNK = 8000
_NCHUNK = _TPTS // _CHUNK
_ITERS = _CHUNK // 16
_CSPLIT = 6


def _make_sc_call(cam_lo, ncams, has_init, do_scale):
    mesh = plsc.VectorSubcoreMesh(core_axis_name="c", subcore_axis_name="s")
    nstep = ncams * _NCHUNK

    @functools.partial(
        pl.kernel,
        out_type=jax.ShapeDtypeStruct((_B * _J * _G3,), jnp.float32),
        mesh=mesh,
        compiler_params=pltpu.CompilerParams(needs_layout_passes=False),
        scratch_types=[
            pltpu.VMEM((_NWORDS,), jnp.int32),
            pltpu.VMEM((_NWORDS,), jnp.int32),
            pltpu.VMEM((_TPTS,), jnp.float32),
            pltpu.VMEM((_CHUNK,), jnp.int32),
            pltpu.VMEM((_CHUNK,), jnp.int32),
            pltpu.SemaphoreType.DMA,
            pltpu.SemaphoreType.DMA,
            pltpu.SemaphoreType.DMA,
            pltpu.SemaphoreType.DMA,
        ],
    )
    def run(*args):
        if has_init:
            hm_hbm, idx_hbm, init_hbm, out_hbm = args[:4]
            rest = args[4:]
        else:
            hm_hbm, idx_hbm, out_hbm = args[:3]
            rest = args[3:]
        (plane_v0, plane_v1, acc_v, idx_v0, idx_v1,
         psem0, psem1, isem0, isem1) = rest
        wid = lax.axis_index("s") * 2 + lax.axis_index("c")
        planes = (plane_v0, plane_v1)
        idxs = (idx_v0, idx_v1)
        psems = (psem0, psem1)
        isems = (isem0, isem1)

        def task(t):
            b = t // (_J * 2)
            rem = t - b * (_J * 2)
            j = rem // 2
            h = rem - j * 2
            out_base = (b * _J + j) * _G3 + h * _TPTS

            def plane_copy(c):
                base = ((b * ncams + c) * _J + j) * _NWORDS
                return pltpu.make_async_copy(
                    hm_hbm.at[pl.ds(base, _NWORDS)],
                    planes[c % 2], psems[c % 2])

            def idx_copy(s):
                c, k = divmod(s, _NCHUNK)
                base = (b * _C + cam_lo + c) * _G3 + h * _TPTS + k * _CHUNK
                return pltpu.make_async_copy(
                    idx_hbm.at[pl.ds(base, _CHUNK)],
                    idxs[s % 2], isems[s % 2])

            plane_copy(0).start()
            idx_copy(0).start()
            if has_init:
                pltpu.sync_copy(init_hbm.at[pl.ds(out_base, _TPTS)], acc_v)
            else:
                @plsc.parallel_loop(0, _TPTS // 16, unroll=4)
                def _zero(i):
                    acc_v[pl.ds(i * 16, 16)] = jnp.zeros((16,), jnp.float32)

            for s in range(nstep):
                c, k = divmod(s, _NCHUNK)
                if s + 1 < nstep:
                    idx_copy(s + 1).start()
                if k == 0:
                    plane_copy(c).wait()
                    if c + 1 < ncams:
                        plane_copy(c + 1).start()
                idx_copy(s).wait()
                pbuf = planes[c % 2]
                ibuf = idxs[s % 2]

                @plsc.parallel_loop(0, _ITERS, unroll=8)
                def _gather(i):
                    iv = ibuf[pl.ds(i * 16, 16)]
                    in_hi = iv >= _NWORDS
                    wi = iv - jnp.where(in_hi, _NWORDS, 0)
                    w = plsc.load_gather(pbuf, [wi])
                    hi = w & jnp.int32(-65536)
                    lo = w << 16
                    bits = jnp.where(in_hi, hi, lo)
                    val = plsc.bitcast(bits, jnp.float32)
                    plsc.addupdate(
                        acc_v.at[pl.ds(k * _CHUNK + i * 16, 16)], val)

            if do_scale:
                @plsc.parallel_loop(0, _TPTS // 16, unroll=4)
                def _scale(i):
                    sl = pl.ds(i * 16, 16)
                    acc_v[sl] = acc_v[sl] * jnp.float32(1.0 / _C)

            pltpu.sync_copy(acc_v, out_hbm.at[pl.ds(out_base, _TPTS)])

        def rounds(r, carry):
            t = wid + r * _NW

            @pl.when(t < _NTASK)
            def _():
                task(t)

            return carry

        lax.fori_loop(0, 3, rounds, 0)

    return run


def _pack(hm):
    u = lax.bitcast_convert_type(hm, jnp.uint32)
    b16 = (u + jnp.uint32(0x7FFF) + ((u >> 16) & jnp.uint32(1))) >> 16
    wlo = b16[:, :, :, : _H // 2, :]
    whi = b16[:, :, :, _H // 2 :, :]
    return lax.bitcast_convert_type(wlo | (whi << 16), jnp.int32).reshape(-1)


def kernel(heatmaps, center, reproLookup):
    cidx = ((center - _OFFSET) / _SPACING).astype(jnp.int32)
    starts = cidx - _HALF

    def slice_b(s):
        return lax.dynamic_slice(
            reproLookup, (jnp.int32(0), s[0], s[1], s[2]), (_C, _G, _G, _G))

    sub_idx = jax.vmap(slice_b)(starts).reshape(_B * _C * _G3)

    part = None
    for lo in range(0, _C, _CSPLIT):
        hm_words = _pack(heatmaps[:, lo:lo + _CSPLIT])
        last = lo + _CSPLIT >= _C
        if part is None:
            part = _make_sc_call(lo, _CSPLIT, False, last)(hm_words, sub_idx)
        else:
            part = _make_sc_call(lo, _CSPLIT, True, last)(
                hm_words, sub_idx, part)
    return part.reshape(_B, _J, _G, _G, _G)
